# baseline (speedup 1.0000x reference)
import jax
import jax.numpy as jnp
from jax import lax
from jax.experimental import pallas as pl
from jax.experimental.pallas import tpu as pltpu
from jax.experimental.pallas import tpu_sc as plsc

N = 10000
E = 320000
D = 128

NC = 2
NS = 16
NW = NC * NS
EPW = E // NW
K = 80
NCHUNK = EPW // K
ZCH = 80
NZC = N // ZCH
MAXM = -(-NZC // NS)


def _chunked(s, fn):
  for m in range(MAXM):
    cid = m * NS + s

    @pl.when(cid < NZC)
    def _():
      fn(cid * ZCH)


def _seg_pass_body(table, src, dst, zeros, out, acc,
                   idx_s0, idx_s1, idx_s2, idx_d0, idx_d1, idx_d2,
                   rows0, rows1, rows2,
                   sem_i0, sem_i1, sem_i2, sem_g0, sem_g1, sem_g2,
                   sem_s0, sem_s1, sem_s2):
  c = lax.axis_index("c")
  s = lax.axis_index("s")
  wid = c * NS + s

  _chunked(s, lambda r0: pltpu.sync_copy(zeros.at[pl.ds(r0, ZCH)],
                                         acc.at[pl.ds(r0, ZCH)]))
  plsc.subcore_barrier()

  base = wid * EPW
  idx_s = (idx_s0, idx_s1, idx_s2)
  idx_d = (idx_d0, idx_d1, idx_d2)
  rows = (rows0, rows1, rows2)
  sem_i = (sem_i0, sem_i1, sem_i2)
  sem_g = (sem_g0, sem_g1, sem_g2)
  sem_s = (sem_s0, sem_s1, sem_s2)

  def issue_idx(g, p):
    off = base + g * K
    pltpu.async_copy(src.at[pl.ds(off, K)], idx_s[p], sem_i[p])
    pltpu.async_copy(dst.at[pl.ds(off, K)], idx_d[p], sem_i[p])

  def wait_idx(g, p):
    off = base + g * K
    pltpu.make_async_copy(src.at[pl.ds(off, K)], idx_s[p], sem_i[p]).wait()
    pltpu.make_async_copy(dst.at[pl.ds(off, K)], idx_d[p], sem_i[p]).wait()

  def start_gather(p):
    pltpu.async_copy(table.at[idx_s[p]], rows[p], sem_g[p])

  def wait_gather(p):
    pltpu.make_async_copy(table.at[idx_s[p]], rows[p], sem_g[p]).wait()

  def fire_scatter(p):
    pltpu.async_copy(rows[p], acc.at[idx_d[p]], sem_s[p], add=True)

  def wait_scatter(p):
    pltpu.make_async_copy(rows[p], acc.at[idx_d[p]], sem_s[p]).wait()

  issue_idx(0, 0)
  wait_idx(0, 0)
  start_gather(0)
  issue_idx(1, 1)

  def slot(g, j):
    @pl.when(g < NCHUNK)
    def _():
      wait_gather(j)
      fire_scatter(j)

    @pl.when((g >= 1) & (g <= NCHUNK))
    def _():
      wait_scatter((j + 2) % 3)

    @pl.when(g + 2 < NCHUNK)
    def _():
      issue_idx(g + 2, (j + 2) % 3)

    @pl.when(g + 1 < NCHUNK)
    def _():
      wait_idx(g + 1, (j + 1) % 3)
      start_gather((j + 1) % 3)

  def body(b, carry):
    for j in range(3):
      slot(3 * b + j, j)
    return carry

  lax.fori_loop(0, (NCHUNK + 3) // 3, body, 0)
  plsc.subcore_barrier()

  _chunked(s, lambda r0: pltpu.sync_copy(acc.at[pl.ds(r0, ZCH)],
                                         out.at[c, pl.ds(r0, ZCH)]))


def _deg_pass_body(dst, zeros, ones80, out, dacc,
                   idx_d0, idx_d1, idx_d2, ones_v,
                   sem_i0, sem_i1, sem_i2, sem_s0, sem_s1, sem_s2):
  c = lax.axis_index("c")
  s = lax.axis_index("s")
  wid = c * NS + s

  pltpu.sync_copy(ones80, ones_v)
  _chunked(s, lambda r0: pltpu.sync_copy(zeros.at[pl.ds(r0, ZCH)],
                                         dacc.at[pl.ds(r0, ZCH)]))
  plsc.subcore_barrier()

  base = wid * EPW
  idx_d = (idx_d0, idx_d1, idx_d2)
  sem_i = (sem_i0, sem_i1, sem_i2)
  sem_s = (sem_s0, sem_s1, sem_s2)

  def issue_idx(g, p):
    pltpu.async_copy(dst.at[pl.ds(base + g * K, K)], idx_d[p], sem_i[p])

  def wait_idx(g, p):
    pltpu.make_async_copy(dst.at[pl.ds(base + g * K, K)], idx_d[p],
                          sem_i[p]).wait()

  def fire_scatter(p):
    pltpu.async_copy(ones_v, dacc.at[idx_d[p]], sem_s[p], add=True)

  def wait_scatter(p):
    pltpu.make_async_copy(ones_v, dacc.at[idx_d[p]], sem_s[p]).wait()

  issue_idx(0, 0)
  issue_idx(1, 1)

  def slot(g, j):
    @pl.when(g < NCHUNK)
    def _():
      wait_idx(g, j)
      fire_scatter(j)

    @pl.when((g >= 1) & (g <= NCHUNK))
    def _():
      wait_scatter((j + 2) % 3)

    @pl.when(g + 2 < NCHUNK)
    def _():
      issue_idx(g + 2, (j + 2) % 3)

  def body(b, carry):
    for j in range(3):
      slot(3 * b + j, j)
    return carry

  lax.fori_loop(0, (NCHUNK + 3) // 3, body, 0)
  plsc.subcore_barrier()

  _chunked(s, lambda r0: pltpu.sync_copy(dacc.at[pl.ds(r0, ZCH)],
                                         out.at[c, pl.ds(r0, ZCH)]))


_SC_MESH = plsc.VectorSubcoreMesh(core_axis_name="c", subcore_axis_name="s")

_seg_pass = pl.kernel(
    _seg_pass_body,
    out_type=jax.ShapeDtypeStruct((NC, N, D), jnp.float32),
    mesh=_SC_MESH,
    scratch_types=(
        [pltpu.VMEM_SHARED((N, D), jnp.float32)]
        + [pltpu.VMEM((K,), jnp.int32)] * 6
        + [pltpu.VMEM((K, D), jnp.float32)] * 3
        + [pltpu.SemaphoreType.DMA] * 9
    ),
)

_deg_pass = pl.kernel(
    _deg_pass_body,
    out_type=jax.ShapeDtypeStruct((NC, N, D), jnp.float32),
    mesh=_SC_MESH,
    scratch_types=(
        [pltpu.VMEM_SHARED((N, D), jnp.float32)]
        + [pltpu.VMEM((K,), jnp.int32)] * 3
        + [pltpu.VMEM((K, D), jnp.float32)]
        + [pltpu.SemaphoreType.DMA] * 6
    ),
)


B1 = 1000
BD = 200


def _layer1_body(f_ref, agg_ref, deg_ref, w_ref, b_ref, h_ref):
  agg = agg_ref[0] + agg_ref[1]
  deg = deg_ref[0, :, 0:1] + deg_ref[1, :, 0:1]
  x = f_ref[...] + agg / jnp.maximum(deg, 1.0)
  y = jnp.dot(x, w_ref[...], preferred_element_type=jnp.float32) + b_ref[...]
  h_ref[...] = jnp.maximum(y, 0.0)


def _layer2_body(h_ref, agg_ref, w_ref, b_ref, z_ref):
  x = h_ref[...] + agg_ref[0] + agg_ref[1]
  y = jnp.dot(x, w_ref[...], preferred_element_type=jnp.float32) + b_ref[...]
  z_ref[...] = jnp.maximum(y, 0.0)


def _decoder_body(zi_ref, zj_ref, o_ref):
  t = lax.dot_general(zi_ref[...], zj_ref[...], (((1,), (1,)), ((), ())),
                      preferred_element_type=jnp.float32)
  o_ref[...] = 1.0 / (1.0 + jnp.exp(-t))


def _layer1(features, agg, deg, w1, b1):
  return pl.pallas_call(
      _layer1_body,
      grid=(N // B1,),
      in_specs=[
          pl.BlockSpec((B1, D), lambda i: (i, 0)),
          pl.BlockSpec((NC, B1, D), lambda i: (0, i, 0)),
          pl.BlockSpec((NC, B1, D), lambda i: (0, i, 0)),
          pl.BlockSpec((D, D), lambda i: (0, 0)),
          pl.BlockSpec((1, D), lambda i: (0, 0)),
      ],
      out_specs=pl.BlockSpec((B1, D), lambda i: (i, 0)),
      out_shape=jax.ShapeDtypeStruct((N, D), jnp.float32),
  )(features, agg, deg, w1, b1)


def _layer2(h, agg, w2p, b2p):
  return pl.pallas_call(
      _layer2_body,
      grid=(N // B1,),
      in_specs=[
          pl.BlockSpec((B1, D), lambda i: (i, 0)),
          pl.BlockSpec((NC, B1, D), lambda i: (0, i, 0)),
          pl.BlockSpec((D, D), lambda i: (0, 0)),
          pl.BlockSpec((1, D), lambda i: (0, 0)),
      ],
      out_specs=pl.BlockSpec((B1, D), lambda i: (i, 0)),
      out_shape=jax.ShapeDtypeStruct((N, D), jnp.float32),
  )(h, agg, w2p, b2p)


def _decoder(z):
  return pl.pallas_call(
      _decoder_body,
      grid=(N // BD,),
      in_specs=[
          pl.BlockSpec((BD, D), lambda i: (i, 0)),
          pl.BlockSpec((N, D), lambda i: (0, 0)),
      ],
      out_specs=pl.BlockSpec((BD, N), lambda i: (i, 0)),
      out_shape=jax.ShapeDtypeStruct((N, N), jnp.float32),
  )(z, z)


@jax.jit
def kernel(features, edge_index, W1, b1, W2, b2):
  src = edge_index[0].astype(jnp.int32)
  dst = edge_index[1].astype(jnp.int32)
  zeros = jnp.zeros((N, D), jnp.float32)
  ones80 = jnp.ones((K, D), jnp.float32)

  agg1 = _seg_pass(features, src, dst, zeros)
  deg = _deg_pass(dst, zeros, ones80)
  h = _layer1(features, agg1, deg, W1, b1.reshape(1, D))

  agg2 = _seg_pass(h, src, dst, zeros)
  w2p = jnp.zeros((D, D), jnp.float32).at[:, :64].set(W2)
  b2p = jnp.zeros((1, D), jnp.float32).at[0, :64].set(b2)
  z = _layer2(h, agg2, w2p, b2p)

  return _decoder(z)


# speedup vs baseline: 1.0031x; 1.0031x over previous
import jax
import jax.numpy as jnp
from jax import lax
from jax.experimental import pallas as pl
from jax.experimental.pallas import tpu as pltpu
from jax.experimental.pallas import tpu_sc as plsc

N = 10000
E = 320000
D = 128

NC = 2
NS = 16
NW = NC * NS
EPW = E // NW
K = 80
NCHUNK = EPW // K
ZCH = 80
NZC = N // ZCH
MAXM = -(-NZC // NS)


def _chunked(s, fn):
  for m in range(MAXM):
    cid = m * NS + s

    @pl.when(cid < NZC)
    def _():
      fn(cid * ZCH)


def _seg_pass_body(table, src, dst, zeros, out, acc,
                   idx_s0, idx_s1, idx_s2, idx_d0, idx_d1, idx_d2,
                   rows0, rows1, rows2,
                   sem_i0, sem_i1, sem_i2, sem_g0, sem_g1, sem_g2,
                   sem_s0, sem_s1, sem_s2):
  c = lax.axis_index("c")
  s = lax.axis_index("s")
  wid = c * NS + s

  _chunked(s, lambda r0: pltpu.sync_copy(zeros.at[pl.ds(r0, ZCH)],
                                         acc.at[pl.ds(r0, ZCH)]))
  plsc.subcore_barrier()

  base = wid * EPW
  idx_s = (idx_s0, idx_s1, idx_s2)
  idx_d = (idx_d0, idx_d1, idx_d2)
  rows = (rows0, rows1, rows2)
  sem_i = (sem_i0, sem_i1, sem_i2)
  sem_g = (sem_g0, sem_g1, sem_g2)
  sem_s = (sem_s0, sem_s1, sem_s2)

  def issue_idx(g, p):
    off = base + g * K
    pltpu.async_copy(src.at[pl.ds(off, K)], idx_s[p], sem_i[p])
    pltpu.async_copy(dst.at[pl.ds(off, K)], idx_d[p], sem_i[p])

  def wait_idx(g, p):
    off = base + g * K
    pltpu.make_async_copy(src.at[pl.ds(off, K)], idx_s[p], sem_i[p]).wait()
    pltpu.make_async_copy(dst.at[pl.ds(off, K)], idx_d[p], sem_i[p]).wait()

  def start_gather(p):
    pltpu.async_copy(table.at[idx_s[p]], rows[p], sem_g[p])

  def wait_gather(p):
    pltpu.make_async_copy(table.at[idx_s[p]], rows[p], sem_g[p]).wait()

  def fire_scatter(p):
    pltpu.async_copy(rows[p], acc.at[idx_d[p]], sem_s[p], add=True)

  def wait_scatter(p):
    pltpu.make_async_copy(rows[p], acc.at[idx_d[p]], sem_s[p]).wait()

  issue_idx(0, 0)
  wait_idx(0, 0)
  start_gather(0)
  issue_idx(1, 1)

  def slot(g, j):
    @pl.when(g < NCHUNK)
    def _():
      wait_gather(j)
      fire_scatter(j)

    @pl.when(g + 1 < NCHUNK)
    def _():
      wait_idx(g + 1, (j + 1) % 3)
      start_gather((j + 1) % 3)

    @pl.when((g >= 1) & (g <= NCHUNK))
    def _():
      wait_scatter((j + 2) % 3)

    @pl.when(g + 2 < NCHUNK)
    def _():
      issue_idx(g + 2, (j + 2) % 3)

  def body(b, carry):
    for j in range(3):
      slot(3 * b + j, j)
    return carry

  lax.fori_loop(0, (NCHUNK + 3) // 3, body, 0)
  plsc.subcore_barrier()

  _chunked(s, lambda r0: pltpu.sync_copy(acc.at[pl.ds(r0, ZCH)],
                                         out.at[c, pl.ds(r0, ZCH)]))


def _deg_pass_body(dst, zeros, ones80, out, dacc,
                   idx_d0, idx_d1, idx_d2, ones_v,
                   sem_i0, sem_i1, sem_i2, sem_s0, sem_s1, sem_s2):
  c = lax.axis_index("c")
  s = lax.axis_index("s")
  wid = c * NS + s

  pltpu.sync_copy(ones80, ones_v)
  _chunked(s, lambda r0: pltpu.sync_copy(zeros.at[pl.ds(r0, ZCH)],
                                         dacc.at[pl.ds(r0, ZCH)]))
  plsc.subcore_barrier()

  base = wid * EPW
  idx_d = (idx_d0, idx_d1, idx_d2)
  sem_i = (sem_i0, sem_i1, sem_i2)
  sem_s = (sem_s0, sem_s1, sem_s2)

  def issue_idx(g, p):
    pltpu.async_copy(dst.at[pl.ds(base + g * K, K)], idx_d[p], sem_i[p])

  def wait_idx(g, p):
    pltpu.make_async_copy(dst.at[pl.ds(base + g * K, K)], idx_d[p],
                          sem_i[p]).wait()

  def fire_scatter(p):
    pltpu.async_copy(ones_v, dacc.at[idx_d[p]], sem_s[p], add=True)

  def wait_scatter(p):
    pltpu.make_async_copy(ones_v, dacc.at[idx_d[p]], sem_s[p]).wait()

  issue_idx(0, 0)
  issue_idx(1, 1)

  def slot(g, j):
    @pl.when(g < NCHUNK)
    def _():
      wait_idx(g, j)
      fire_scatter(j)

    @pl.when((g >= 1) & (g <= NCHUNK))
    def _():
      wait_scatter((j + 2) % 3)

    @pl.when(g + 2 < NCHUNK)
    def _():
      issue_idx(g + 2, (j + 2) % 3)

  def body(b, carry):
    for j in range(3):
      slot(3 * b + j, j)
    return carry

  lax.fori_loop(0, (NCHUNK + 3) // 3, body, 0)
  plsc.subcore_barrier()

  _chunked(s, lambda r0: pltpu.sync_copy(dacc.at[pl.ds(r0, ZCH)],
                                         out.at[c, pl.ds(r0, ZCH)]))


_SC_MESH = plsc.VectorSubcoreMesh(core_axis_name="c", subcore_axis_name="s")

_seg_pass = pl.kernel(
    _seg_pass_body,
    out_type=jax.ShapeDtypeStruct((NC, N, D), jnp.float32),
    mesh=_SC_MESH,
    scratch_types=(
        [pltpu.VMEM_SHARED((N, D), jnp.float32)]
        + [pltpu.VMEM((K,), jnp.int32)] * 6
        + [pltpu.VMEM((K, D), jnp.float32)] * 3
        + [pltpu.SemaphoreType.DMA] * 9
    ),
)

_deg_pass = pl.kernel(
    _deg_pass_body,
    out_type=jax.ShapeDtypeStruct((NC, N, D), jnp.float32),
    mesh=_SC_MESH,
    scratch_types=(
        [pltpu.VMEM_SHARED((N, D), jnp.float32)]
        + [pltpu.VMEM((K,), jnp.int32)] * 3
        + [pltpu.VMEM((K, D), jnp.float32)]
        + [pltpu.SemaphoreType.DMA] * 6
    ),
)


B1 = 1000
BD = 200


def _layer1_body(f_ref, agg_ref, deg_ref, w_ref, b_ref, h_ref):
  agg = agg_ref[0] + agg_ref[1]
  deg = deg_ref[0, :, 0:1] + deg_ref[1, :, 0:1]
  x = f_ref[...] + agg / jnp.maximum(deg, 1.0)
  y = jnp.dot(x, w_ref[...], preferred_element_type=jnp.float32) + b_ref[...]
  h_ref[...] = jnp.maximum(y, 0.0)


def _layer2_body(h_ref, agg_ref, w_ref, b_ref, z_ref):
  x = h_ref[...] + agg_ref[0] + agg_ref[1]
  y = jnp.dot(x, w_ref[...], preferred_element_type=jnp.float32) + b_ref[...]
  z_ref[...] = jnp.maximum(y, 0.0)


def _decoder_body(zi_ref, zj_ref, o_ref):
  t = lax.dot_general(zi_ref[...], zj_ref[...], (((1,), (1,)), ((), ())),
                      preferred_element_type=jnp.float32)
  o_ref[...] = 1.0 / (1.0 + jnp.exp(-t))


def _layer1(features, agg, deg, w1, b1):
  return pl.pallas_call(
      _layer1_body,
      grid=(N // B1,),
      in_specs=[
          pl.BlockSpec((B1, D), lambda i: (i, 0)),
          pl.BlockSpec((NC, B1, D), lambda i: (0, i, 0)),
          pl.BlockSpec((NC, B1, D), lambda i: (0, i, 0)),
          pl.BlockSpec((D, D), lambda i: (0, 0)),
          pl.BlockSpec((1, D), lambda i: (0, 0)),
      ],
      out_specs=pl.BlockSpec((B1, D), lambda i: (i, 0)),
      out_shape=jax.ShapeDtypeStruct((N, D), jnp.float32),
  )(features, agg, deg, w1, b1)


def _layer2(h, agg, w2p, b2p):
  return pl.pallas_call(
      _layer2_body,
      grid=(N // B1,),
      in_specs=[
          pl.BlockSpec((B1, D), lambda i: (i, 0)),
          pl.BlockSpec((NC, B1, D), lambda i: (0, i, 0)),
          pl.BlockSpec((D, D), lambda i: (0, 0)),
          pl.BlockSpec((1, D), lambda i: (0, 0)),
      ],
      out_specs=pl.BlockSpec((B1, D), lambda i: (i, 0)),
      out_shape=jax.ShapeDtypeStruct((N, D), jnp.float32),
  )(h, agg, w2p, b2p)


def _decoder(z):
  return pl.pallas_call(
      _decoder_body,
      grid=(N // BD,),
      in_specs=[
          pl.BlockSpec((BD, D), lambda i: (i, 0)),
          pl.BlockSpec((N, D), lambda i: (0, 0)),
      ],
      out_specs=pl.BlockSpec((BD, N), lambda i: (i, 0)),
      out_shape=jax.ShapeDtypeStruct((N, N), jnp.float32),
  )(z, z)


@jax.jit
def kernel(features, edge_index, W1, b1, W2, b2):
  src = edge_index[0].astype(jnp.int32)
  dst = edge_index[1].astype(jnp.int32)
  zeros = jnp.zeros((N, D), jnp.float32)
  ones80 = jnp.ones((K, D), jnp.float32)

  agg1 = _seg_pass(features, src, dst, zeros)
  deg = _deg_pass(dst, zeros, ones80)
  h = _layer1(features, agg1, deg, W1, b1.reshape(1, D))

  agg2 = _seg_pass(h, src, dst, zeros)
  w2p = jnp.zeros((D, D), jnp.float32).at[:, :64].set(W2)
  b2p = jnp.zeros((1, D), jnp.float32).at[0, :64].set(b2)
  z = _layer2(h, agg2, w2p, b2p)

  return _decoder(z)


# speedup vs baseline: 1.0503x; 1.0470x over previous
import jax
import jax.numpy as jnp
from jax import lax
from jax.experimental import pallas as pl
from jax.experimental.pallas import tpu as pltpu
from jax.experimental.pallas import tpu_sc as plsc

N = 10000
E = 320000
D = 128

NC = 2
NS = 16
NW = NC * NS
EPW = E // NW
K = 80
NCHUNK = EPW // K
ZCH = 80
NZC = N // ZCH
MAXM = -(-NZC // NS)


def _chunked(s, fn):
  for m in range(MAXM):
    cid = m * NS + s

    @pl.when(cid < NZC)
    def _():
      fn(cid * ZCH)


def _seg_pass_body(table, src, dst, zeros, out, acc,
                   idx_s0, idx_s1, idx_d0, idx_d1, rows0, rows1,
                   sem_i0, sem_i1, sem_g0, sem_g1):
  c = lax.axis_index("c")
  s = lax.axis_index("s")
  wid = c * NS + s

  _chunked(s, lambda r0: pltpu.sync_copy(zeros.at[pl.ds(r0, ZCH)],
                                         acc.at[pl.ds(r0, ZCH)]))
  plsc.subcore_barrier()

  base = wid * EPW
  idx_s = (idx_s0, idx_s1)
  idx_d = (idx_d0, idx_d1)
  rows = (rows0, rows1)
  sem_i = (sem_i0, sem_i1)
  sem_g = (sem_g0, sem_g1)

  def issue_idx(g, p):
    off = base + g * K
    pltpu.async_copy(src.at[pl.ds(off, K)], idx_s[p], sem_i[p])
    pltpu.async_copy(dst.at[pl.ds(off, K)], idx_d[p], sem_i[p])

  def wait_idx(g, p):
    off = base + g * K
    pltpu.make_async_copy(src.at[pl.ds(off, K)], idx_s[p], sem_i[p]).wait()
    pltpu.make_async_copy(dst.at[pl.ds(off, K)], idx_d[p], sem_i[p]).wait()

  def start_gather(p):
    pltpu.async_copy(table.at[idx_s[p]], rows[p], sem_g[p])

  def wait_gather(p):
    pltpu.make_async_copy(table.at[idx_s[p]], rows[p], sem_g[p]).wait()

  def scatter(p):
    pltpu.sync_copy(rows[p], acc.at[idx_d[p]], add=True)

  issue_idx(0, 0)
  wait_idx(0, 0)
  start_gather(0)
  issue_idx(1, 1)

  def body(b, carry):
    g1 = 2 * b + 1
    g2 = 2 * b + 2
    g3 = 2 * b + 3

    @pl.when(g1 < NCHUNK)
    def _():
      wait_idx(g1, 1)
      start_gather(1)
    wait_gather(0)
    scatter(0)

    @pl.when(g2 < NCHUNK)
    def _():
      issue_idx(g2, 0)

    @pl.when(g1 < NCHUNK)
    def _():
      @pl.when(g2 < NCHUNK)
      def _():
        wait_idx(g2, 0)
        start_gather(0)
      wait_gather(1)
      scatter(1)

      @pl.when(g3 < NCHUNK)
      def _():
        issue_idx(g3, 1)
    return carry

  lax.fori_loop(0, (NCHUNK + 1) // 2, body, 0)
  plsc.subcore_barrier()

  _chunked(s, lambda r0: pltpu.sync_copy(acc.at[pl.ds(r0, ZCH)],
                                         out.at[c, pl.ds(r0, ZCH)]))


def _deg_pass_body(dst, zeros, ones80, out, dacc, idx_d0, idx_d1, ones_v,
                   sem_i0, sem_i1):
  c = lax.axis_index("c")
  s = lax.axis_index("s")
  wid = c * NS + s

  pltpu.sync_copy(ones80, ones_v)
  _chunked(s, lambda r0: pltpu.sync_copy(zeros.at[pl.ds(r0, ZCH)],
                                         dacc.at[pl.ds(r0, ZCH)]))
  plsc.subcore_barrier()

  base = wid * EPW
  idx_d = (idx_d0, idx_d1)
  sem_i = (sem_i0, sem_i1)

  def issue_idx(g, p):
    pltpu.async_copy(dst.at[pl.ds(base + g * K, K)], idx_d[p], sem_i[p])

  def wait_idx(g, p):
    pltpu.make_async_copy(dst.at[pl.ds(base + g * K, K)], idx_d[p],
                          sem_i[p]).wait()

  def scatter(p):
    pltpu.sync_copy(ones_v, dacc.at[idx_d[p]], add=True)

  issue_idx(0, 0)
  issue_idx(1, 1)

  def body(b, carry):
    g1 = 2 * b + 1
    g2 = 2 * b + 2
    g3 = 2 * b + 3
    wait_idx(2 * b, 0)
    scatter(0)

    @pl.when(g2 < NCHUNK)
    def _():
      issue_idx(g2, 0)

    @pl.when(g1 < NCHUNK)
    def _():
      wait_idx(g1, 1)
      scatter(1)

      @pl.when(g3 < NCHUNK)
      def _():
        issue_idx(g3, 1)
    return carry

  lax.fori_loop(0, (NCHUNK + 1) // 2, body, 0)
  plsc.subcore_barrier()

  _chunked(s, lambda r0: pltpu.sync_copy(dacc.at[pl.ds(r0, ZCH)],
                                         out.at[c, pl.ds(r0, ZCH)]))


_SC_MESH = plsc.VectorSubcoreMesh(core_axis_name="c", subcore_axis_name="s")

_seg_pass = pl.kernel(
    _seg_pass_body,
    out_type=jax.ShapeDtypeStruct((NC, N, D), jnp.float32),
    mesh=_SC_MESH,
    scratch_types=(
        [pltpu.VMEM_SHARED((N, D), jnp.float32)]
        + [pltpu.VMEM((K,), jnp.int32)] * 4
        + [pltpu.VMEM((K, D), jnp.float32)] * 2
        + [pltpu.SemaphoreType.DMA] * 4
    ),
)

_deg_pass = pl.kernel(
    _deg_pass_body,
    out_type=jax.ShapeDtypeStruct((NC, N, D), jnp.float32),
    mesh=_SC_MESH,
    scratch_types=(
        [pltpu.VMEM_SHARED((N, D), jnp.float32)]
        + [pltpu.VMEM((K,), jnp.int32)] * 2
        + [pltpu.VMEM((K, D), jnp.float32)]
        + [pltpu.SemaphoreType.DMA] * 2
    ),
)


B1 = 1000
BD = 400


def _layer1_body(f_ref, agg_ref, deg_ref, w_ref, b_ref, h_ref):
  agg = agg_ref[0] + agg_ref[1]
  deg = deg_ref[0, :, 0:1] + deg_ref[1, :, 0:1]
  x = f_ref[...] + agg / jnp.maximum(deg, 1.0)
  y = jnp.dot(x, w_ref[...], preferred_element_type=jnp.float32) + b_ref[...]
  h_ref[...] = jnp.maximum(y, 0.0)


def _layer2_body(h_ref, agg_ref, w_ref, b_ref, z_ref):
  x = h_ref[...] + agg_ref[0] + agg_ref[1]
  y = jnp.dot(x, w_ref[...], preferred_element_type=jnp.float32) + b_ref[...]
  z_ref[...] = jnp.maximum(y, 0.0)


def _decoder_body(zi_ref, zj_ref, o_ref):
  t = lax.dot_general(zi_ref[...], zj_ref[...], (((1,), (1,)), ((), ())),
                      preferred_element_type=jnp.float32)
  o_ref[...] = 1.0 / (1.0 + jnp.exp(-t))


def _layer1(features, agg, deg, w1, b1):
  return pl.pallas_call(
      _layer1_body,
      grid=(N // B1,),
      in_specs=[
          pl.BlockSpec((B1, D), lambda i: (i, 0)),
          pl.BlockSpec((NC, B1, D), lambda i: (0, i, 0)),
          pl.BlockSpec((NC, B1, D), lambda i: (0, i, 0)),
          pl.BlockSpec((D, D), lambda i: (0, 0)),
          pl.BlockSpec((1, D), lambda i: (0, 0)),
      ],
      out_specs=pl.BlockSpec((B1, D), lambda i: (i, 0)),
      out_shape=jax.ShapeDtypeStruct((N, D), jnp.float32),
  )(features, agg, deg, w1, b1)


def _layer2(h, agg, w2p, b2p):
  return pl.pallas_call(
      _layer2_body,
      grid=(N // B1,),
      in_specs=[
          pl.BlockSpec((B1, D), lambda i: (i, 0)),
          pl.BlockSpec((NC, B1, D), lambda i: (0, i, 0)),
          pl.BlockSpec((D, D), lambda i: (0, 0)),
          pl.BlockSpec((1, D), lambda i: (0, 0)),
      ],
      out_specs=pl.BlockSpec((B1, D), lambda i: (i, 0)),
      out_shape=jax.ShapeDtypeStruct((N, D), jnp.float32),
  )(h, agg, w2p, b2p)


def _decoder(z):
  return pl.pallas_call(
      _decoder_body,
      grid=(N // BD,),
      in_specs=[
          pl.BlockSpec((BD, D), lambda i: (i, 0)),
          pl.BlockSpec((N, D), lambda i: (0, 0)),
      ],
      out_specs=pl.BlockSpec((BD, N), lambda i: (i, 0)),
      out_shape=jax.ShapeDtypeStruct((N, N), jnp.float32),
  )(z, z)


@jax.jit
def kernel(features, edge_index, W1, b1, W2, b2):
  src = edge_index[0].astype(jnp.int32)
  dst = edge_index[1].astype(jnp.int32)
  zeros = jnp.zeros((N, D), jnp.float32)
  ones80 = jnp.ones((K, D), jnp.float32)

  agg1 = _seg_pass(features, src, dst, zeros)
  deg = _deg_pass(dst, zeros, ones80)
  h = _layer1(features, agg1, deg, W1, b1.reshape(1, D))

  agg2 = _seg_pass(h, src, dst, zeros)
  w2p = jnp.zeros((D, D), jnp.float32).at[:, :64].set(W2)
  b2p = jnp.zeros((1, D), jnp.float32).at[0, :64].set(b2)
  z = _layer2(h, agg2, w2p, b2p)

  return _decoder(z)
